# Initial kernel scaffold; baseline (speedup 1.0000x reference)
import jax
import jax.numpy as jnp
from jax.experimental import pallas as pl


def kernel(flat_pts, batch_idx, W1, b1, W2, b2):
    raise NotImplementedError("write your pallas kernel here")


# speedup vs baseline: 4.3099x; 4.3099x over previous
import functools

import jax
import jax.numpy as jnp
from jax import lax
from jax.experimental import pallas as pl
from jax.experimental.pallas import tpu as pltpu

N = 32768
B = 16
IN_DIM = 64
HIDDEN = 512
OUT4 = 256 * 4

TILE = 1024
NTILES = N // TILE

_NEG = float("-inf")


def _body(offs_s, lo_s, hi_s, x_ref, w1_ref, b1_ref, w2_ref, b2_ref,
          out_ref, pooled_ref):
    i = pl.program_id(0)

    @pl.when(i == 0)
    def _init():
        pooled_ref[...] = jnp.full((B, HIDDEN), _NEG, dtype=jnp.float32)

    h = jnp.dot(x_ref[...], w1_ref[...],
                preferred_element_type=jnp.float32) + b1_ref[...]

    rowid = lax.broadcasted_iota(jnp.int32, (TILE, 1), 0) + i * TILE
    lo = lo_s[i]
    hi = hi_s[i]

    def seg_step(s, carry):
        m = (rowid >= offs_s[s]) & (rowid < offs_s[s + 1])
        seg = jnp.max(jnp.where(m, h, _NEG), axis=0, keepdims=True)
        pooled_ref[pl.ds(s, 1), :] = jnp.maximum(pooled_ref[pl.ds(s, 1), :], seg)
        return carry

    lax.fori_loop(lo, hi + 1, seg_step, 0)

    @pl.when(i == NTILES - 1)
    def _finish():
        out_ref[...] = jnp.dot(pooled_ref[...], w2_ref[...],
                               preferred_element_type=jnp.float32) + b2_ref[...]


@jax.jit
def _encode(flat_pts, batch_idx, W1, b1, W2, b2):
    idx = batch_idx.astype(jnp.int32)
    offs = jnp.searchsorted(idx, jnp.arange(B + 1, dtype=jnp.int32)).astype(jnp.int32)
    starts = jnp.arange(NTILES, dtype=jnp.int32) * TILE
    tile_lo = idx[::TILE]
    tile_hi = idx[TILE - 1::TILE]
    del starts

    grid_spec = pltpu.PrefetchScalarGridSpec(
        num_scalar_prefetch=3,
        grid=(NTILES,),
        in_specs=[
            pl.BlockSpec((TILE, IN_DIM), lambda i, *_: (i, 0)),
            pl.BlockSpec((IN_DIM, HIDDEN), lambda i, *_: (0, 0)),
            pl.BlockSpec((1, HIDDEN), lambda i, *_: (0, 0)),
            pl.BlockSpec((HIDDEN, OUT4), lambda i, *_: (0, 0)),
            pl.BlockSpec((1, OUT4), lambda i, *_: (0, 0)),
        ],
        out_specs=pl.BlockSpec((B, OUT4), lambda i, *_: (0, 0)),
        scratch_shapes=[pltpu.VMEM((B, HIDDEN), jnp.float32)],
    )

    proj = pl.pallas_call(
        _body,
        grid_spec=grid_spec,
        out_shape=jax.ShapeDtypeStruct((B, OUT4), jnp.float32),
        compiler_params=pltpu.CompilerParams(
            dimension_semantics=("arbitrary",),
        ),
    )(offs, tile_lo, tile_hi,
      flat_pts, W1, b1.reshape(1, HIDDEN), W2, b2.reshape(1, OUT4))
    return proj.reshape(B, OUT4 // 4, 4)


def kernel(flat_pts, batch_idx, W1, b1, W2, b2):
    return _encode(flat_pts, batch_idx, W1, b1, W2, b2)


# speedup vs baseline: 5.8303x; 1.3528x over previous
import functools

import jax
import jax.numpy as jnp
from jax import lax
from jax.experimental import pallas as pl
from jax.experimental.pallas import tpu as pltpu

N = 32768
B = 16
IN_DIM = 64
HIDDEN = 512
OUT4 = 256 * 4

TILE = 1024
NTILES = N // TILE

_NEG = float("-inf")


def _body(offs_s, lo_s, hi_s, x_ref, w1_ref, b1_ref, w2_ref, b2_ref,
          out_ref, pooled_ref):
    i = pl.program_id(0)

    @pl.when(i == 0)
    def _init():
        pooled_ref[...] = jnp.full((B, HIDDEN), _NEG, dtype=jnp.float32)

    h = jnp.dot(x_ref[...].astype(jnp.bfloat16), w1_ref[...].astype(jnp.bfloat16),
                preferred_element_type=jnp.float32) + b1_ref[...]

    rowid = lax.broadcasted_iota(jnp.int32, (TILE, 1), 0) + i * TILE
    lo = lo_s[i]
    hi = hi_s[i]

    def seg_step(s, carry):
        m = (rowid >= offs_s[s]) & (rowid < offs_s[s + 1])
        seg = jnp.max(jnp.where(m, h, _NEG), axis=0, keepdims=True)
        pooled_ref[pl.ds(s, 1), :] = jnp.maximum(pooled_ref[pl.ds(s, 1), :], seg)
        return carry

    lax.fori_loop(lo, hi + 1, seg_step, 0)

    @pl.when(i == NTILES - 1)
    def _finish():
        out_ref[...] = jnp.dot(pooled_ref[...], w2_ref[...],
                               preferred_element_type=jnp.float32) + b2_ref[...]


@jax.jit
def _encode(flat_pts, batch_idx, W1, b1, W2, b2):
    idx = batch_idx.astype(jnp.int32)
    offs = jnp.sum(idx[:, None] < jnp.arange(B + 1, dtype=jnp.int32)[None, :],
                   axis=0, dtype=jnp.int32)
    tile_lo = idx[::TILE]
    tile_hi = idx[TILE - 1::TILE]

    grid_spec = pltpu.PrefetchScalarGridSpec(
        num_scalar_prefetch=3,
        grid=(NTILES,),
        in_specs=[
            pl.BlockSpec((TILE, IN_DIM), lambda i, *_: (i, 0)),
            pl.BlockSpec((IN_DIM, HIDDEN), lambda i, *_: (0, 0)),
            pl.BlockSpec((1, HIDDEN), lambda i, *_: (0, 0)),
            pl.BlockSpec((HIDDEN, OUT4), lambda i, *_: (0, 0)),
            pl.BlockSpec((1, OUT4), lambda i, *_: (0, 0)),
        ],
        out_specs=pl.BlockSpec((B, OUT4), lambda i, *_: (0, 0)),
        scratch_shapes=[pltpu.VMEM((B, HIDDEN), jnp.float32)],
    )

    proj = pl.pallas_call(
        _body,
        grid_spec=grid_spec,
        out_shape=jax.ShapeDtypeStruct((B, OUT4), jnp.float32),
        compiler_params=pltpu.CompilerParams(
            dimension_semantics=("arbitrary",),
        ),
    )(offs, tile_lo, tile_hi,
      flat_pts, W1, b1.reshape(1, HIDDEN), W2, b2.reshape(1, OUT4))
    return proj.reshape(B, OUT4 // 4, 4)


def kernel(flat_pts, batch_idx, W1, b1, W2, b2):
    return _encode(flat_pts, batch_idx, W1, b1, W2, b2)
